# baseline (speedup 1.0000x reference)
import functools

import jax
import jax.numpy as jnp
from jax.experimental import pallas as pl


def _gat_kernel(x_ref, adj_ref, w_ref, asrc_ref, adst_ref, bias_ref, out_ref,
                *, dst_chunk):
    N = adj_ref.shape[1]
    C = dst_chunk
    H = w_ref.shape[1]
    j = pl.program_id(1)

    x = x_ref[0]
    xp = jax.lax.dot(x, w_ref[...], preferred_element_type=jnp.float32)

    s_src = jax.lax.dot_general(
        xp, asrc_ref[...], (((1,), (1,)), ((), ())),
        preferred_element_type=jnp.float32)
    xj = x_ref[0, pl.ds(j * C, C), :]
    xpj = jax.lax.dot(xj, w_ref[...], preferred_element_type=jnp.float32)
    s_dst = jax.lax.dot_general(
        adst_ref[...], xpj, (((1,), (1,)), ((), ())),
        preferred_element_type=jnp.float32)

    s_max = jnp.max(s_src)
    mrow = s_max + s_dst
    mrow = jnp.maximum(mrow, 0.2 * mrow)

    logits = s_src + s_dst
    logits = jnp.maximum(logits, 0.2 * logits)

    row = jax.lax.broadcasted_iota(jnp.int32, (N, C), 0)
    col = jax.lax.broadcasted_iota(jnp.int32, (N, C), 1) + j * C
    mask = (adj_ref[0] != 0) | (row == col)
    e = jnp.where(mask, jnp.exp2(logits - mrow), 0.0)

    ones = jnp.ones((N, 1), dtype=jnp.float32)
    xp_aug = jnp.concatenate([xp, ones], axis=1)
    num_aug = jax.lax.dot_general(
        e.astype(jnp.bfloat16), xp_aug.astype(jnp.bfloat16),
        (((0,), (0,)), ((), ())),
        preferred_element_type=jnp.float32)

    denom = num_aug[:, H:H + 1] + 1e-16
    out_ref[0] = num_aug[:, :H] / denom + bias_ref[...]


@jax.jit
def kernel(x, adj_matrix, W, a_src, a_dst, bias):
    B, S, N, F = x.shape
    H = W.shape[1]
    T = B * S
    K = 4
    C = N // K

    xf = x.reshape(T, N, F)
    adjf = adj_matrix.reshape(T, N, N)
    log2e = jnp.float32(1.4426950408889634)
    a_src2 = (a_src * log2e).reshape(1, H)
    a_dst2 = (a_dst * log2e).reshape(1, H)
    bias2 = bias.reshape(1, H)

    out = pl.pallas_call(
        functools.partial(_gat_kernel, dst_chunk=C),
        grid=(T, K),
        in_specs=[
            pl.BlockSpec((1, N, F), lambda t, j: (t, 0, 0)),
            pl.BlockSpec((1, N, C), lambda t, j: (t, 0, j)),
            pl.BlockSpec((F, H), lambda t, j: (0, 0)),
            pl.BlockSpec((1, H), lambda t, j: (0, 0)),
            pl.BlockSpec((1, H), lambda t, j: (0, 0)),
            pl.BlockSpec((1, H), lambda t, j: (0, 0)),
        ],
        out_specs=pl.BlockSpec((1, C, H), lambda t, j: (t, j, 0)),
        out_shape=jax.ShapeDtypeStruct((T, N, H), jnp.float32),
    )(xf, adjf, W, a_src2, a_dst2, bias2)

    return out.reshape(B, S, N, H)


# speedup vs baseline: 1.0574x; 1.0574x over previous
import functools

import jax
import jax.numpy as jnp
from jax.experimental import pallas as pl
from jax.experimental.pallas import tpu as pltpu


def _gat_kernel(x_ref, adj_ref, w_ref, asrc_ref, adst_ref, bias_ref, out_ref,
                acc_ref, *, src_chunk, num_chunks):
    R = src_chunk
    N = x_ref.shape[1]
    H = w_ref.shape[1]
    j = pl.program_id(1)

    x = x_ref[0]
    xp = jax.lax.dot(x, w_ref[...], preferred_element_type=jnp.float32)

    s_src = jax.lax.dot_general(
        xp, asrc_ref[...], (((1,), (1,)), ((), ())),
        preferred_element_type=jnp.float32)
    s_dst = jax.lax.dot_general(
        adst_ref[...], xp, (((1,), (1,)), ((), ())),
        preferred_element_type=jnp.float32)

    s_max = jnp.max(s_src)
    mrow = s_max + s_dst
    mrow = jnp.maximum(mrow, 0.2 * mrow)

    xj = x_ref[0, pl.ds(j * R, R), :]
    xp_j = jax.lax.dot(xj, w_ref[...], preferred_element_type=jnp.float32)
    s_src_j = jax.lax.dot_general(
        xp_j, asrc_ref[...], (((1,), (1,)), ((), ())),
        preferred_element_type=jnp.float32)
    logits = s_src_j + s_dst
    logits = jnp.maximum(logits, 0.2 * logits)

    row = jax.lax.broadcasted_iota(jnp.int32, (R, N), 0) + j * R
    col = jax.lax.broadcasted_iota(jnp.int32, (R, N), 1)
    mask = (adj_ref[0] != 0) | (row == col)
    e = jnp.where(mask, jnp.exp2(logits - mrow), 0.0)

    ones = jnp.ones((R, 1), dtype=jnp.float32)
    xp_aug = jnp.concatenate([xp_j, ones], axis=1)
    part = jax.lax.dot_general(
        e.astype(jnp.bfloat16), xp_aug.astype(jnp.bfloat16),
        (((0,), (0,)), ((), ())),
        preferred_element_type=jnp.float32)

    @pl.when(j == 0)
    def _init():
        acc_ref[...] = part

    @pl.when(j > 0)
    def _accum():
        acc_ref[...] += part

    @pl.when(j == num_chunks - 1)
    def _finalize():
        acc = acc_ref[...]
        denom = acc[:, H:H + 1] + 1e-16
        out_ref[0] = acc[:, :H] / denom + bias_ref[...]


@jax.jit
def kernel(x, adj_matrix, W, a_src, a_dst, bias):
    B, S, N, F = x.shape
    H = W.shape[1]
    T = B * S
    K = 4
    R = N // K

    xf = x.reshape(T, N, F)
    adjf = adj_matrix.reshape(T, N, N)
    log2e = jnp.float32(1.4426950408889634)
    a_src2 = (a_src * log2e).reshape(1, H)
    a_dst2 = (a_dst * log2e).reshape(1, H)
    bias2 = bias.reshape(1, H)

    out = pl.pallas_call(
        functools.partial(_gat_kernel, src_chunk=R, num_chunks=K),
        grid=(T, K),
        in_specs=[
            pl.BlockSpec((1, N, F), lambda t, j: (t, 0, 0)),
            pl.BlockSpec((1, R, N), lambda t, j: (t, j, 0)),
            pl.BlockSpec((F, H), lambda t, j: (0, 0)),
            pl.BlockSpec((1, H), lambda t, j: (0, 0)),
            pl.BlockSpec((1, H), lambda t, j: (0, 0)),
            pl.BlockSpec((1, H), lambda t, j: (0, 0)),
        ],
        out_specs=pl.BlockSpec((1, N, H), lambda t, j: (t, 0, 0)),
        out_shape=jax.ShapeDtypeStruct((T, N, H), jnp.float32),
        scratch_shapes=[pltpu.VMEM((N, H + 1), jnp.float32)],
    )(xf, adjf, W, a_src2, a_dst2, bias2)

    return out.reshape(B, S, N, H)


# speedup vs baseline: 1.7722x; 1.6759x over previous
import functools

import jax
import jax.numpy as jnp
from jax.experimental import pallas as pl


def _gat_kernel(x_ref, adj_ref, w_ref, asrc_ref, adst_ref, bias_ref, out_ref):
    N = adj_ref.shape[1]
    H = w_ref.shape[1]

    x = x_ref[0]
    xp = jax.lax.dot(x, w_ref[...], preferred_element_type=jnp.float32)

    s_src = jax.lax.dot_general(
        xp, asrc_ref[...], (((1,), (1,)), ((), ())),
        preferred_element_type=jnp.float32)
    s_dst = jax.lax.dot_general(
        adst_ref[...], xp, (((1,), (1,)), ((), ())),
        preferred_element_type=jnp.float32)

    s_max = jnp.max(s_src)
    mrow = s_max + s_dst
    mrow = jnp.maximum(mrow, 0.2 * mrow)

    logits = s_src + s_dst
    logits = jnp.maximum(logits, 0.2 * logits)

    row = jax.lax.broadcasted_iota(jnp.int32, (N, N), 0)
    col = jax.lax.broadcasted_iota(jnp.int32, (N, N), 1)
    mask = (adj_ref[0] != 0) | (row == col)
    e = jnp.where(mask, jnp.exp2(logits - mrow), 0.0)

    ones = jnp.ones((N, 1), dtype=jnp.float32)
    xp_aug = jnp.concatenate([xp, ones], axis=1)
    num_aug = jax.lax.dot_general(
        e.astype(jnp.bfloat16), xp_aug.astype(jnp.bfloat16),
        (((0,), (0,)), ((), ())),
        preferred_element_type=jnp.float32)

    denom = num_aug[:, H:H + 1] + 1e-16
    out_ref[0] = num_aug[:, :H] / denom + bias_ref[...]


@jax.jit
def kernel(x, adj_matrix, W, a_src, a_dst, bias):
    B, S, N, F = x.shape
    H = W.shape[1]
    T = B * S

    xf = x.reshape(T, N, F)
    adjf = adj_matrix.reshape(T, N, N)
    log2e = jnp.float32(1.4426950408889634)
    a_src2 = (a_src * log2e).reshape(1, H)
    a_dst2 = (a_dst * log2e).reshape(1, H)
    bias2 = bias.reshape(1, H)

    out = pl.pallas_call(
        _gat_kernel,
        grid=(T,),
        in_specs=[
            pl.BlockSpec((1, N, F), lambda t: (t, 0, 0)),
            pl.BlockSpec((1, N, N), lambda t: (t, 0, 0)),
            pl.BlockSpec((F, H), lambda t: (0, 0)),
            pl.BlockSpec((1, H), lambda t: (0, 0)),
            pl.BlockSpec((1, H), lambda t: (0, 0)),
            pl.BlockSpec((1, H), lambda t: (0, 0)),
        ],
        out_specs=pl.BlockSpec((1, N, H), lambda t: (t, 0, 0)),
        out_shape=jax.ShapeDtypeStruct((T, N, H), jnp.float32),
    )(xf, adjf, W, a_src2, a_dst2, bias2)

    return out.reshape(B, S, N, H)


# speedup vs baseline: 1.8470x; 1.0422x over previous
import jax
import jax.numpy as jnp
from jax.experimental import pallas as pl

_LOG2E = 1.4426950408889634


def _gat_kernel(x_ref, adj_ref, w_ref, asrc_ref, adst_ref, bias_ref, out_ref):
    N = adj_ref.shape[2]
    H = w_ref.shape[1]

    x = x_ref[0, 0]
    xp = jax.lax.dot(x, w_ref[...], preferred_element_type=jnp.float32)

    s_src = jax.lax.dot_general(
        xp, asrc_ref[...], (((1,), (1,)), ((), ())),
        preferred_element_type=jnp.float32) * _LOG2E
    s_dst = jax.lax.dot_general(
        adst_ref[...], xp, (((1,), (1,)), ((), ())),
        preferred_element_type=jnp.float32) * _LOG2E

    s_max = jnp.max(s_src)
    mrow = s_max + s_dst
    mrow = jnp.maximum(mrow, 0.2 * mrow)

    logits = s_src + s_dst
    logits = jnp.maximum(logits, 0.2 * logits)

    row = jax.lax.broadcasted_iota(jnp.int32, (N, N), 0)
    col = jax.lax.broadcasted_iota(jnp.int32, (N, N), 1)
    mask = (adj_ref[0, 0] != 0) | (row == col)
    e = jnp.where(mask, jnp.exp2(logits - mrow), 0.0)

    ones = jnp.ones((N, 1), dtype=jnp.float32)
    xp_aug = jnp.concatenate([xp, ones], axis=1)
    num_aug = jax.lax.dot_general(
        e.astype(jnp.bfloat16), xp_aug.astype(jnp.bfloat16),
        (((0,), (0,)), ((), ())),
        preferred_element_type=jnp.float32)

    denom = num_aug[:, H:H + 1] + 1e-16
    out_ref[0, 0] = num_aug[:, :H] / denom + bias_ref[...]


@jax.jit
def kernel(x, adj_matrix, W, a_src, a_dst, bias):
    B, S, N, F = x.shape
    H = W.shape[1]

    a_src2 = a_src.reshape(1, H)
    a_dst2 = a_dst.reshape(1, H)
    bias2 = bias.reshape(1, H)

    return pl.pallas_call(
        _gat_kernel,
        grid=(B, S),
        in_specs=[
            pl.BlockSpec((1, 1, N, F), lambda b, s: (b, s, 0, 0)),
            pl.BlockSpec((1, 1, N, N), lambda b, s: (b, s, 0, 0)),
            pl.BlockSpec((F, H), lambda b, s: (0, 0)),
            pl.BlockSpec((1, H), lambda b, s: (0, 0)),
            pl.BlockSpec((1, H), lambda b, s: (0, 0)),
            pl.BlockSpec((1, H), lambda b, s: (0, 0)),
        ],
        out_specs=pl.BlockSpec((1, 1, N, H), lambda b, s: (b, s, 0, 0)),
        out_shape=jax.ShapeDtypeStruct((B, S, N, H), jnp.float32),
    )(x, adj_matrix, W, a_src2, a_dst2, bias2)


# speedup vs baseline: 1.9061x; 1.0320x over previous
import functools

import jax
import jax.numpy as jnp
from jax.experimental import pallas as pl
from jax.experimental.layout import Format, Layout

_LOG2E = 1.4426950408889634


def _gat_kernel(xp_ref, adj_ref, asrc_ref, adst_ref, bias_ref, out_ref):
    N = adj_ref.shape[2]
    H = xp_ref.shape[3]

    xp = xp_ref[0, 0]

    s_src = jax.lax.dot_general(
        xp, asrc_ref[...], (((1,), (1,)), ((), ())),
        preferred_element_type=jnp.float32) * _LOG2E
    s_dst = jax.lax.dot_general(
        adst_ref[...], xp, (((1,), (1,)), ((), ())),
        preferred_element_type=jnp.float32) * _LOG2E

    s_max = jnp.max(s_src)
    mrow = s_max + s_dst
    mrow = jnp.maximum(mrow, 0.2 * mrow)

    logits = s_src + s_dst
    logits = jnp.maximum(logits, 0.2 * logits)

    row = jax.lax.broadcasted_iota(jnp.int32, (N, N), 0)
    col = jax.lax.broadcasted_iota(jnp.int32, (N, N), 1)
    mask = (adj_ref[0, 0] != 0) | (row == col)
    e = jnp.where(mask, jnp.exp2(logits - mrow), 0.0)

    ones = jnp.ones((N, 1), dtype=jnp.float32)
    xp_aug = jnp.concatenate([xp, ones], axis=1)
    num_aug = jax.lax.dot_general(
        e.astype(jnp.bfloat16), xp_aug.astype(jnp.bfloat16),
        (((0,), (0,)), ((), ())),
        preferred_element_type=jnp.float32)

    denom = num_aug[:, H:H + 1] + 1e-16
    out_ref[0, 0] = num_aug[:, :H] / denom + bias_ref[...]


def _gat(x, adj_matrix, W, a_src, a_dst, bias):
    B, S, N, F = x.shape
    H = W.shape[1]

    xp = jnp.einsum("bsnf,fh->bsnh", x, W,
                    preferred_element_type=jnp.float32)
    a_src2 = a_src.reshape(1, H)
    a_dst2 = a_dst.reshape(1, H)
    bias2 = bias.reshape(1, H)

    return pl.pallas_call(
        _gat_kernel,
        grid=(B, S),
        in_specs=[
            pl.BlockSpec((1, 1, N, H), lambda b, s: (b, s, 0, 0)),
            pl.BlockSpec((1, 1, N, N), lambda b, s: (b, s, 0, 0)),
            pl.BlockSpec((1, H), lambda b, s: (0, 0)),
            pl.BlockSpec((1, H), lambda b, s: (0, 0)),
            pl.BlockSpec((1, H), lambda b, s: (0, 0)),
        ],
        out_specs=pl.BlockSpec((1, 1, N, H), lambda b, s: (b, s, 0, 0)),
        out_shape=jax.ShapeDtypeStruct((B, S, N, H), jnp.float32),
    )(xp, adj_matrix, a_src2, a_dst2, bias2)


_jitted = None


def kernel(x, adj_matrix, W, a_src, a_dst, bias):
    global _jitted
    if _jitted is None:
        try:
            fmt = Format(Layout(major_to_minor=(3, 2, 1, 0), tiling=((8, 128),)),
                         jax.sharding.SingleDeviceSharding(jax.devices()[0]))
            _jitted = jax.jit(_gat, out_shardings=fmt)
        except Exception:
            _jitted = jax.jit(_gat)
    return _jitted(x, adj_matrix, W, a_src, a_dst, bias)
